# baseline (speedup 1.0000x reference)
import functools

import jax
import jax.numpy as jnp
from jax import lax
from jax.experimental import pallas as pl
from jax.experimental.pallas import tpu as pltpu
from jax.experimental.pallas import tpu_sc as plsc

_N_CORES = 2
_N_SUB = 16
_G = 64
_C = 128
_CW = 16

_TC_PARAMS = pltpu.CompilerParams(vmem_limit_bytes=100 * 1024 * 1024)


def _dot(a, b, precision=None):
    return jax.lax.dot_general(a, b, (((1,), (0,)), ((), ())),
                               precision=precision,
                               preferred_element_type=jnp.float32)


@functools.lru_cache(maxsize=None)
def _make_sc_pass(N8, NCH, W, with_counts):
    C = _C
    RPT = (N8 // _N_SUB) // 8 * 8
    TAIL = N8 - _N_SUB * RPT
    NBUF = 5
    ZR = 104
    assert TAIL % 8 == 0 and 0 <= TAIL <= ZR
    assert NCH % NBUF == 0 and NCH // NBUF >= 2
    assert RPT % ZR == 0

    mesh = plsc.VectorSubcoreMesh(core_axis_name="c", subcore_axis_name="s")

    out_type = [jax.ShapeDtypeStruct((_N_CORES, N8, W), jnp.float32)]
    scratch = [
        pltpu.VMEM((NCH, C), jnp.int32),
        pltpu.VMEM((NCH, C), jnp.int32),
        pltpu.VMEM((NBUF, C, W), jnp.float32),
        pltpu.VMEM((ZR, W), jnp.float32),
        pltpu.VMEM_SHARED((N8, W), jnp.float32),
    ]
    if with_counts:
        out_type.append(
            jax.ShapeDtypeStruct((_N_CORES, N8, _CW), jnp.float32))
        scratch.append(pltpu.VMEM((C, _CW), jnp.float32))
        scratch.append(
            pltpu.VMEM_SHARED((N8, _CW), jnp.float32))
    nsem = NBUF + 2 + (1 if with_counts else 0)
    scratch += [pltpu.SemaphoreType.DMA] * nsem

    def sc_pass_body(u_hbm, src_hbm, dst_hbm, *rest):
        if with_counts:
            (out_hbm, outc_hbm, sidx, didx, rows, zbuf, acc, ones, accc,
             *sems) = rest
        else:
            (out_hbm, sidx, didx, rows, zbuf, acc, *sems) = rest
            outc_hbm = ones = accc = None
        csem = sems[NBUF + 2] if with_counts else None

        c = lax.axis_index("c")
        s = lax.axis_index("s")
        w = c * _N_SUB + s

        pltpu.async_copy(src_hbm.at[w], sidx, sems[NBUF])
        pltpu.async_copy(dst_hbm.at[w], didx, sems[NBUF + 1])

        zero = jnp.zeros((16,), jnp.float32)
        one = jnp.ones((16,), jnp.float32)

        def zrow(i, carry):
            for k in range(W // 16):
                zbuf[i, pl.ds(k * 16, 16)] = zero
            return carry

        lax.fori_loop(0, ZR, zrow, 0)
        if with_counts:
            def orow(i, carry):
                ones[i, pl.ds(0, 16)] = one
                return carry
            lax.fori_loop(0, C, orow, 0)

        pltpu.make_async_copy(src_hbm.at[w], sidx, sems[NBUF]).wait()

        def gather_start(i, j):
            pltpu.async_copy(u_hbm.at[sidx.at[i]], rows.at[j], sems[j])

        def gather_wait(j):
            pltpu.make_async_copy(u_hbm.at[pl.ds(0, C)], rows.at[j],
                                  sems[j]).wait()

        for j in range(NBUF):
            gather_start(j, j)

        for k in range(RPT // ZR):
            pltpu.sync_copy(zbuf, acc.at[pl.ds(s * RPT + k * ZR, ZR)])
        if TAIL:
            @pl.when(s == _N_SUB - 1)
            def _():
                pltpu.sync_copy(zbuf.at[pl.ds(0, TAIL)],
                                acc.at[pl.ds(_N_SUB * RPT, TAIL)])
        if with_counts:
            zc = zbuf.at[pl.ds(0, ZR), pl.ds(0, _CW)]
            for k in range(RPT // ZR):
                pltpu.sync_copy(zc, accc.at[pl.ds(s * RPT + k * ZR, ZR)])
            if TAIL:
                @pl.when(s == _N_SUB - 1)
                def _():
                    pltpu.sync_copy(zbuf.at[pl.ds(0, TAIL), pl.ds(0, _CW)],
                                    accc.at[pl.ds(_N_SUB * RPT, TAIL)])

        pltpu.make_async_copy(dst_hbm.at[w], didx, sems[NBUF + 1]).wait()
        plsc.subcore_barrier()

        def chunk(i, j, refill):
            gather_wait(j)
            pltpu.sync_copy(rows.at[j], acc.at[didx.at[i]], add=True)
            if with_counts:
                pltpu.async_copy(ones, accc.at[didx.at[i]], csem, add=True)
            if refill:
                gather_start(i + NBUF, j)

        def body(g, carry):
            for j in range(NBUF):
                chunk(g * NBUF + j, j, True)
            return carry

        lax.fori_loop(0, NCH // NBUF - 1, body, 0)
        for j in range(NBUF):
            chunk(NCH - NBUF + j, j, False)

        if with_counts:
            def drain(i, carry):
                pltpu.make_async_copy(ones, accc.at[didx.at[0]],
                                      csem).wait()
                return carry
            lax.fori_loop(0, NCH, drain, 0)

        plsc.subcore_barrier()
        pltpu.sync_copy(acc.at[pl.ds(s * RPT, RPT)],
                        out_hbm.at[c, pl.ds(s * RPT, RPT)])
        if with_counts:
            pltpu.sync_copy(accc.at[pl.ds(s * RPT, RPT)],
                            outc_hbm.at[c, pl.ds(s * RPT, RPT)])
        if TAIL:
            @pl.when(s == _N_SUB - 1)
            def _():
                pltpu.sync_copy(acc.at[pl.ds(_N_SUB * RPT, TAIL)],
                                out_hbm.at[c, pl.ds(_N_SUB * RPT, TAIL)])
                if with_counts:
                    pltpu.sync_copy(accc.at[pl.ds(_N_SUB * RPT, TAIL)],
                                    outc_hbm.at[c, pl.ds(_N_SUB * RPT, TAIL)])

    return pl.kernel(
        sc_pass_body,
        out_type=out_type,
        mesh=mesh,
        scratch_types=scratch,
        compiler_params=pltpu.CompilerParams(use_tc_tiling_on_sc=False),
    )


def _tc_pre_body(x_ref, wl_ref, wr_ref, u_ref, r_ref):
    x = x_ref[...]
    u_ref[...] = _dot(x, wl_ref[...])
    r_ref[...] = _dot(x, wr_ref[...])


def _tc_mid_body(n_real, sa_ref, sb_ref, ca_ref, cb_ref, r1_ref, b_ref,
                 wr_ref, h1_ref, r2_ref):
    ssum = sa_ref[...] + sb_ref[...]
    cnt = ca_ref[:, :1] + cb_ref[:, :1]
    inv = 1.0 / jnp.maximum(cnt, 1.0)
    h1 = jnp.maximum(ssum * inv + b_ref[...] + r1_ref[...], 0.0)
    n8 = h1.shape[0]
    mask = (lax.broadcasted_iota(jnp.int32, (n8, 1), 0)
            < n_real).astype(jnp.float32)
    h1 = h1 * mask
    h1_ref[...] = h1
    r2 = _dot(h1, wr_ref[...])
    r2_ref[...] = jnp.concatenate(
        [r2, jnp.broadcast_to(inv, r2.shape)], axis=1)


def _tc_post_body(sa_ref, sb_ref, r2_ref, b_ref, bt_ref, wl_ref,
                  wa1_ref, ba1_ref, wa2_ref, ba2_ref,
                  wc1_ref, bc1_ref, wc2_ref, bc2_ref,
                  mu_ref, val_ref):
    r2a = r2_ref[...]
    h = sa_ref.shape[1]
    inv = r2a[:, h:h + 1]
    mean2 = (sa_ref[...] + sb_ref[...]) * inv
    h2 = jnp.maximum(_dot(mean2, wl_ref[...]) + b_ref[...] + r2a[:, :h],
                     0.0)
    n = h2.shape[0]
    bt = jnp.broadcast_to(bt_ref[...], (_G, n))
    ohT = (bt == lax.broadcasted_iota(jnp.int32, (_G, n), 0)) \
        .astype(jnp.float32)
    sums = _dot(ohT, h2, precision=jax.lax.Precision.HIGHEST)
    cnts = jnp.sum(ohT, axis=1, keepdims=True)
    pooled = sums / jnp.maximum(cnts, 1.0)
    a = jnp.maximum(_dot(pooled, wa1_ref[...]) + ba1_ref[...], 0.0)
    mu_ref[...] = _dot(a, wa2_ref[...]) + ba2_ref[...]
    cv = jnp.maximum(_dot(pooled, wc1_ref[...]) + bc1_ref[...], 0.0)
    val_ref[...] = _dot(cv, wc2_ref[...]) + bc2_ref[...]


def kernel(x, edge_index, batch, W1l, b1l, W1r, W2l, b2l, W2r,
           Wa1, ba1, Wa2, ba2, Wc1, bc1, Wc2, bc2):
    N, F = x.shape
    E = edge_index.shape[1]
    H = W1l.shape[0]
    A = Wa2.shape[0]
    N8 = N + 8
    NW = _N_CORES * _N_SUB
    NCH = -(-E // (NW * _C * 5)) * 5
    EPAD = NW * NCH * _C

    f32 = jnp.float32
    i32 = jnp.int32
    pad = jnp.full((EPAD - E,), N, i32)
    src3 = jnp.concatenate([edge_index[0], pad]).reshape(NW, NCH, _C)
    dst3 = jnp.concatenate([edge_index[1], pad]).reshape(NW, NCH, _C)
    x8 = jnp.pad(x, ((0, N8 - N), (0, 0)))
    batch8 = jnp.pad(batch, (0, N8 - N), constant_values=_G)

    u1, r1 = pl.pallas_call(
        _tc_pre_body,
        out_shape=[jax.ShapeDtypeStruct((N8, H), f32),
                   jax.ShapeDtypeStruct((N8, H), f32)],
        compiler_params=_TC_PARAMS,
    )(x8, W1l.T, W1r.T)

    s1, c1 = _make_sc_pass(N8, NCH, H, True)(u1, src3, dst3)

    h1, r2a = pl.pallas_call(
        functools.partial(_tc_mid_body, N),
        out_shape=[jax.ShapeDtypeStruct((N8, H), f32),
                   jax.ShapeDtypeStruct((N8, 2 * H), f32)],
        compiler_params=_TC_PARAMS,
    )(s1[0], s1[1], c1[0], c1[1], r1, b1l.reshape(1, H), W2r.T)

    s2, = _make_sc_pass(N8, NCH, H, False)(h1, src3, dst3)

    mu, value = pl.pallas_call(
        _tc_post_body,
        out_shape=[jax.ShapeDtypeStruct((_G, A), f32),
                   jax.ShapeDtypeStruct((_G, 1), f32)],
        compiler_params=_TC_PARAMS,
    )(s2[0], s2[1], r2a, b2l.reshape(1, H), batch8.reshape(1, N8), W2l.T,
      Wa1.T, ba1.reshape(1, H), Wa2.T, ba2.reshape(1, A),
      Wc1.T, bc1.reshape(1, H), Wc2.T, bc2.reshape(1, 1))

    return (mu, value)


# speedup vs baseline: 2.5203x; 2.5203x over previous
import functools

import jax
import jax.numpy as jnp
from jax import lax
from jax.experimental import pallas as pl
from jax.experimental.pallas import tpu as pltpu
from jax.experimental.pallas import tpu_sc as plsc

_N_CORES = 2
_N_SUB = 16
_G = 64
_C = 128
_CW = 16

_TC_PARAMS = pltpu.CompilerParams(vmem_limit_bytes=100 * 1024 * 1024)


def _dot(a, b, precision=None):
    return jax.lax.dot_general(a, b, (((1,), (0,)), ((), ())),
                               precision=precision,
                               preferred_element_type=jnp.float32)


@functools.lru_cache(maxsize=None)
def _make_sc_pass(N8, NCH, W, with_counts):
    C = _C
    RPT = (N8 // _N_SUB) // 8 * 8
    TAIL = N8 - _N_SUB * RPT
    NBUF = 5
    ZR = 160
    assert TAIL % 8 == 0 and 0 <= TAIL <= ZR
    assert NCH % NBUF == 0 and NCH // NBUF >= 2
    assert RPT % ZR == 0

    mesh = plsc.VectorSubcoreMesh(core_axis_name="c", subcore_axis_name="s")

    out_type = [jax.ShapeDtypeStruct((_N_CORES, N8, W), jnp.float32)]
    scratch = [
        pltpu.VMEM((NCH, C), jnp.int32),
        pltpu.VMEM((NCH, C), jnp.int32),
        pltpu.VMEM((NBUF, C, W), jnp.float32),
        pltpu.VMEM((ZR, W), jnp.float32),
        pltpu.VMEM_SHARED((N8, W), jnp.float32),
    ]
    if with_counts:
        out_type.append(
            jax.ShapeDtypeStruct((_N_CORES, N8, _CW), jnp.float32))
        scratch.append(pltpu.VMEM((C, _CW), jnp.float32))
        scratch.append(
            pltpu.VMEM_SHARED((N8, _CW), jnp.float32))
    nsem = NBUF + 2 + (1 if with_counts else 0)
    scratch += [pltpu.SemaphoreType.DMA] * nsem

    def sc_pass_body(u_hbm, src_hbm, dst_hbm, *rest):
        if with_counts:
            (out_hbm, outc_hbm, sidx, didx, rows, zbuf, acc, ones, accc,
             *sems) = rest
        else:
            (out_hbm, sidx, didx, rows, zbuf, acc, *sems) = rest
            outc_hbm = ones = accc = None
        csem = sems[NBUF + 2] if with_counts else None

        c = lax.axis_index("c")
        s = lax.axis_index("s")
        w = c * _N_SUB + s

        pltpu.async_copy(src_hbm.at[w], sidx, sems[NBUF])
        pltpu.async_copy(dst_hbm.at[w], didx, sems[NBUF + 1])

        zero = jnp.zeros((16,), jnp.float32)
        one = jnp.ones((16,), jnp.float32)

        def zrow(i, carry):
            for k in range(W // 16):
                zbuf[i, pl.ds(k * 16, 16)] = zero
            return carry

        lax.fori_loop(0, ZR, zrow, 0)
        if with_counts:
            def orow(i, carry):
                ones[i, pl.ds(0, 16)] = one
                return carry
            lax.fori_loop(0, C, orow, 0)

        pltpu.make_async_copy(src_hbm.at[w], sidx, sems[NBUF]).wait()

        def gather_start(i, j):
            pltpu.async_copy(u_hbm.at[sidx.at[i]], rows.at[j], sems[j])

        def gather_wait(j):
            pltpu.make_async_copy(u_hbm.at[pl.ds(0, C)], rows.at[j],
                                  sems[j]).wait()

        for j in range(NBUF):
            gather_start(j, j)

        for k in range(RPT // ZR):
            pltpu.sync_copy(zbuf, acc.at[pl.ds(s * RPT + k * ZR, ZR)])
        if TAIL:
            @pl.when(s == _N_SUB - 1)
            def _():
                pltpu.sync_copy(zbuf.at[pl.ds(0, TAIL)],
                                acc.at[pl.ds(_N_SUB * RPT, TAIL)])
        if with_counts:
            zc = zbuf.at[pl.ds(0, ZR), pl.ds(0, _CW)]
            for k in range(RPT // ZR):
                pltpu.sync_copy(zc, accc.at[pl.ds(s * RPT + k * ZR, ZR)])
            if TAIL:
                @pl.when(s == _N_SUB - 1)
                def _():
                    pltpu.sync_copy(zbuf.at[pl.ds(0, TAIL), pl.ds(0, _CW)],
                                    accc.at[pl.ds(_N_SUB * RPT, TAIL)])

        pltpu.make_async_copy(dst_hbm.at[w], didx, sems[NBUF + 1]).wait()
        plsc.subcore_barrier()

        def chunk(i, j, refill):
            gather_wait(j)
            pltpu.sync_copy(rows.at[j], acc.at[didx.at[i]], add=True)
            if with_counts:
                pltpu.async_copy(ones, accc.at[didx.at[i]], csem, add=True)
            if refill:
                gather_start(i + NBUF, j)

        def body(g, carry):
            for j in range(NBUF):
                chunk(g * NBUF + j, j, True)
            return carry

        lax.fori_loop(0, NCH // NBUF - 1, body, 0)
        for j in range(NBUF):
            chunk(NCH - NBUF + j, j, False)

        if with_counts:
            def drain(i, carry):
                pltpu.make_async_copy(ones, accc.at[didx.at[0]],
                                      csem).wait()
                return carry
            lax.fori_loop(0, NCH, drain, 0)

        plsc.subcore_barrier()
        pltpu.sync_copy(acc.at[pl.ds(s * RPT, RPT)],
                        out_hbm.at[c, pl.ds(s * RPT, RPT)])
        if with_counts:
            pltpu.sync_copy(accc.at[pl.ds(s * RPT, RPT)],
                            outc_hbm.at[c, pl.ds(s * RPT, RPT)])
        if TAIL:
            @pl.when(s == _N_SUB - 1)
            def _():
                pltpu.sync_copy(acc.at[pl.ds(_N_SUB * RPT, TAIL)],
                                out_hbm.at[c, pl.ds(_N_SUB * RPT, TAIL)])
                if with_counts:
                    pltpu.sync_copy(accc.at[pl.ds(_N_SUB * RPT, TAIL)],
                                    outc_hbm.at[c, pl.ds(_N_SUB * RPT, TAIL)])

    return pl.kernel(
        sc_pass_body,
        out_type=out_type,
        mesh=mesh,
        scratch_types=scratch,
        compiler_params=pltpu.CompilerParams(use_tc_tiling_on_sc=False),
    )


def _tc_pre_body(x_ref, wl_ref, wr_ref, u_ref, r_ref):
    x = x_ref[...]
    u_ref[...] = _dot(x, wl_ref[...])
    r_ref[...] = _dot(x, wr_ref[...])


def _tc_mid_body(n_real, sa_ref, sb_ref, ca_ref, cb_ref, r1_ref, b_ref,
                 wr_ref, h1_ref, r2_ref):
    ssum = sa_ref[...] + sb_ref[...]
    cnt = ca_ref[:, :1] + cb_ref[:, :1]
    inv = 1.0 / jnp.maximum(cnt, 1.0)
    h1 = jnp.maximum(ssum * inv + b_ref[...] + r1_ref[...], 0.0)
    n8 = h1.shape[0]
    mask = (lax.broadcasted_iota(jnp.int32, (n8, 1), 0)
            < n_real).astype(jnp.float32)
    h1 = h1 * mask
    h1_ref[...] = h1
    r2 = _dot(h1, wr_ref[...])
    r2_ref[...] = jnp.concatenate(
        [r2, jnp.broadcast_to(inv, r2.shape)], axis=1)


def _tc_post_body(sa_ref, sb_ref, r2_ref, b_ref, bt_ref, wl_ref,
                  wa1_ref, ba1_ref, wa2_ref, ba2_ref,
                  wc1_ref, bc1_ref, wc2_ref, bc2_ref,
                  mu_ref, val_ref):
    r2a = r2_ref[...]
    h = sa_ref.shape[1]
    inv = r2a[:, h:h + 1]
    mean2 = (sa_ref[...] + sb_ref[...]) * inv
    h2 = jnp.maximum(_dot(mean2, wl_ref[...]) + b_ref[...] + r2a[:, :h],
                     0.0)
    n = h2.shape[0]
    bt = jnp.broadcast_to(bt_ref[...], (_G, n))
    ohT = (bt == lax.broadcasted_iota(jnp.int32, (_G, n), 0)) \
        .astype(jnp.float32)
    sums = _dot(ohT, h2, precision=jax.lax.Precision.HIGHEST)
    cnts = jnp.sum(ohT, axis=1, keepdims=True)
    pooled = sums / jnp.maximum(cnts, 1.0)
    a = jnp.maximum(_dot(pooled, wa1_ref[...]) + ba1_ref[...], 0.0)
    mu_ref[...] = _dot(a, wa2_ref[...]) + ba2_ref[...]
    cv = jnp.maximum(_dot(pooled, wc1_ref[...]) + bc1_ref[...], 0.0)
    val_ref[...] = _dot(cv, wc2_ref[...]) + bc2_ref[...]


def kernel(x, edge_index, batch, W1l, b1l, W1r, W2l, b2l, W2r,
           Wa1, ba1, Wa2, ba2, Wc1, bc1, Wc2, bc2):
    N, F = x.shape
    E = edge_index.shape[1]
    H = W1l.shape[0]
    A = Wa2.shape[0]
    NDUM = 240
    N8 = N + NDUM
    NW = _N_CORES * _N_SUB
    NCH = -(-E // (NW * _C * 5)) * 5
    EPAD = NW * NCH * _C

    f32 = jnp.float32
    i32 = jnp.int32
    pad = N + jnp.arange(EPAD - E, dtype=i32) % NDUM
    src3 = jnp.concatenate([edge_index[0], pad]).reshape(NW, NCH, _C)
    dst3 = jnp.concatenate([edge_index[1], pad]).reshape(NW, NCH, _C)
    x8 = jnp.pad(x, ((0, N8 - N), (0, 0)))
    batch8 = jnp.pad(batch, (0, N8 - N), constant_values=_G)

    u1, r1 = pl.pallas_call(
        _tc_pre_body,
        out_shape=[jax.ShapeDtypeStruct((N8, H), f32),
                   jax.ShapeDtypeStruct((N8, H), f32)],
        compiler_params=_TC_PARAMS,
    )(x8, W1l.T, W1r.T)

    s1, c1 = _make_sc_pass(N8, NCH, H, True)(u1, src3, dst3)

    h1, r2a = pl.pallas_call(
        functools.partial(_tc_mid_body, N),
        out_shape=[jax.ShapeDtypeStruct((N8, H), f32),
                   jax.ShapeDtypeStruct((N8, 2 * H), f32)],
        compiler_params=_TC_PARAMS,
    )(s1[0], s1[1], c1[0], c1[1], r1, b1l.reshape(1, H), W2r.T)

    s2, = _make_sc_pass(N8, NCH, H, False)(h1, src3, dst3)

    mu, value = pl.pallas_call(
        _tc_post_body,
        out_shape=[jax.ShapeDtypeStruct((_G, A), f32),
                   jax.ShapeDtypeStruct((_G, 1), f32)],
        compiler_params=_TC_PARAMS,
    )(s2[0], s2[1], r2a, b2l.reshape(1, H), batch8.reshape(1, N8), W2l.T,
      Wa1.T, ba1.reshape(1, H), Wa2.T, ba2.reshape(1, A),
      Wc1.T, bc1.reshape(1, H), Wc2.T, bc2.reshape(1, 1))

    return (mu, value)


# speedup vs baseline: 2.7920x; 1.1078x over previous
import functools

import jax
import jax.numpy as jnp
from jax import lax
from jax.experimental import pallas as pl
from jax.experimental.pallas import tpu as pltpu
from jax.experimental.pallas import tpu_sc as plsc

_N_CORES = 2
_N_SUB = 16
_G = 64
_C = 128
_CW = 16

_TC_PARAMS = pltpu.CompilerParams(vmem_limit_bytes=100 * 1024 * 1024)


def _dot(a, b, precision=None):
    return jax.lax.dot_general(a, b, (((1,), (0,)), ((), ())),
                               precision=precision,
                               preferred_element_type=jnp.float32)


@functools.lru_cache(maxsize=None)
def _make_sc_pass(N8, NCH, W, with_counts):
    C = _C
    RPT = (N8 // _N_SUB) // 8 * 8
    TAIL = N8 - _N_SUB * RPT
    NBUF = 5
    ZR = 160
    assert TAIL % 8 == 0 and 0 <= TAIL <= ZR
    assert NCH % NBUF == 0 and NCH // NBUF >= 2
    assert RPT % ZR == 0

    mesh = plsc.VectorSubcoreMesh(core_axis_name="c", subcore_axis_name="s")

    out_type = [jax.ShapeDtypeStruct((_N_CORES, N8, W), jnp.float32)]
    scratch = [
        pltpu.VMEM((NCH, C), jnp.int32),
        pltpu.VMEM((NCH, C), jnp.int32),
        pltpu.VMEM((NBUF, C, W), jnp.float32),
        pltpu.VMEM((ZR, W), jnp.float32),
        pltpu.VMEM_SHARED((N8, W), jnp.float32),
    ]
    if with_counts:
        out_type.append(
            jax.ShapeDtypeStruct((_N_CORES, N8, _CW), jnp.float32))
        scratch.append(pltpu.VMEM((C, _CW), jnp.float32))
        scratch.append(
            pltpu.VMEM_SHARED((N8, _CW), jnp.float32))
    nsem = NBUF + 2 + (1 if with_counts else 0)
    scratch += [pltpu.SemaphoreType.DMA] * nsem

    def sc_pass_body(u_hbm, src_hbm, dst_hbm, *rest):
        if with_counts:
            (out_hbm, outc_hbm, sidx, didx, rows, zbuf, acc, ones, accc,
             *sems) = rest
        else:
            (out_hbm, sidx, didx, rows, zbuf, acc, *sems) = rest
            outc_hbm = ones = accc = None
        csem = sems[NBUF + 2] if with_counts else None

        c = lax.axis_index("c")
        s = lax.axis_index("s")
        w = c * _N_SUB + s

        pltpu.async_copy(src_hbm.at[w], sidx, sems[NBUF])
        pltpu.async_copy(dst_hbm.at[w], didx, sems[NBUF + 1])

        zero = jnp.zeros((16,), jnp.float32)
        one = jnp.ones((16,), jnp.float32)

        def zrow(i, carry):
            for k in range(W // 16):
                zbuf[i, pl.ds(k * 16, 16)] = zero
            return carry

        lax.fori_loop(0, ZR, zrow, 0)
        if with_counts:
            def orow(i, carry):
                ones[i, pl.ds(0, 16)] = one
                return carry
            lax.fori_loop(0, C, orow, 0)

        pltpu.make_async_copy(src_hbm.at[w], sidx, sems[NBUF]).wait()

        def gather_start(i, j):
            pltpu.async_copy(u_hbm.at[sidx.at[i]], rows.at[j], sems[j])

        def gather_wait(j):
            pltpu.make_async_copy(u_hbm.at[pl.ds(0, C)], rows.at[j],
                                  sems[j]).wait()

        for j in range(NBUF):
            gather_start(j, j)

        for k in range(RPT // ZR):
            pltpu.sync_copy(zbuf, acc.at[pl.ds(s * RPT + k * ZR, ZR)])
        if TAIL:
            @pl.when(s == _N_SUB - 1)
            def _():
                pltpu.sync_copy(zbuf.at[pl.ds(0, TAIL)],
                                acc.at[pl.ds(_N_SUB * RPT, TAIL)])
        if with_counts:
            zc = zbuf.at[pl.ds(0, ZR), pl.ds(0, _CW)]
            for k in range(RPT // ZR):
                pltpu.sync_copy(zc, accc.at[pl.ds(s * RPT + k * ZR, ZR)])
            if TAIL:
                @pl.when(s == _N_SUB - 1)
                def _():
                    pltpu.sync_copy(zbuf.at[pl.ds(0, TAIL), pl.ds(0, _CW)],
                                    accc.at[pl.ds(_N_SUB * RPT, TAIL)])

        pltpu.make_async_copy(dst_hbm.at[w], didx, sems[NBUF + 1]).wait()
        plsc.subcore_barrier()

        def chunk(i, j, refill):
            gather_wait(j)
            pltpu.sync_copy(rows.at[j], acc.at[didx.at[i]], add=True)
            if with_counts:
                pltpu.async_copy(ones, accc.at[didx.at[i]], csem, add=True)
            if refill:
                gather_start(i + NBUF, j)

        def body(g, carry):
            for j in range(NBUF):
                chunk(g * NBUF + j, j, True)
            return carry

        lax.fori_loop(0, NCH // NBUF - 1, body, 0)
        for j in range(NBUF):
            chunk(NCH - NBUF + j, j, False)

        if with_counts:
            def drain(i, carry):
                pltpu.make_async_copy(ones, accc.at[didx.at[0]],
                                      csem).wait()
                return carry
            lax.fori_loop(0, NCH, drain, 0)

        plsc.subcore_barrier()
        pltpu.sync_copy(acc.at[pl.ds(s * RPT, RPT)],
                        out_hbm.at[c, pl.ds(s * RPT, RPT)])
        if with_counts:
            pltpu.sync_copy(accc.at[pl.ds(s * RPT, RPT)],
                            outc_hbm.at[c, pl.ds(s * RPT, RPT)])
        if TAIL:
            @pl.when(s == _N_SUB - 1)
            def _():
                pltpu.sync_copy(acc.at[pl.ds(_N_SUB * RPT, TAIL)],
                                out_hbm.at[c, pl.ds(_N_SUB * RPT, TAIL)])
                if with_counts:
                    pltpu.sync_copy(accc.at[pl.ds(_N_SUB * RPT, TAIL)],
                                    outc_hbm.at[c, pl.ds(_N_SUB * RPT, TAIL)])

    return pl.kernel(
        sc_pass_body,
        out_type=out_type,
        mesh=mesh,
        scratch_types=scratch,
        compiler_params=pltpu.CompilerParams(use_tc_tiling_on_sc=False),
    )


def _tc_pre_body(x_ref, wl_ref, wr_ref, u_ref, r_ref):
    x = x_ref[...]
    u_ref[...] = _dot(x, wl_ref[...])
    r_ref[...] = _dot(x, wr_ref[...])


def _tc_mid_body(n_real, s_ref, c_ref, r1_ref, b_ref,
                 wr_ref, h1_ref, r2_ref):
    ssum = s_ref[0] + s_ref[1]
    cnt = c_ref[0][:, :1] + c_ref[1][:, :1]
    inv = 1.0 / jnp.maximum(cnt, 1.0)
    h1 = jnp.maximum(ssum * inv + b_ref[...] + r1_ref[...], 0.0)
    n8 = h1.shape[0]
    mask = (lax.broadcasted_iota(jnp.int32, (n8, 1), 0)
            < n_real).astype(jnp.float32)
    h1 = h1 * mask
    h1_ref[...] = h1
    r2 = _dot(h1, wr_ref[...])
    r2_ref[...] = jnp.concatenate(
        [r2, jnp.broadcast_to(inv, r2.shape)], axis=1)


def _tc_post_body(s_ref, r2_ref, b_ref, bt_ref, wl_ref,
                  wa1_ref, ba1_ref, wa2_ref, ba2_ref,
                  wc1_ref, bc1_ref, wc2_ref, bc2_ref,
                  mu_ref, val_ref):
    r2a = r2_ref[...]
    h = s_ref.shape[2]
    inv = r2a[:, h:h + 1]
    mean2 = (s_ref[0] + s_ref[1]) * inv
    h2 = jnp.maximum(_dot(mean2, wl_ref[...]) + b_ref[...] + r2a[:, :h],
                     0.0)
    n = h2.shape[0]
    bt = jnp.broadcast_to(bt_ref[...], (_G, n))
    ohT = (bt == lax.broadcasted_iota(jnp.int32, (_G, n), 0)) \
        .astype(jnp.float32)
    sums = _dot(ohT, h2, precision=jax.lax.Precision.HIGHEST)
    cnts = jnp.sum(ohT, axis=1, keepdims=True)
    pooled = sums / jnp.maximum(cnts, 1.0)
    a = jnp.maximum(_dot(pooled, wa1_ref[...]) + ba1_ref[...], 0.0)
    mu_ref[...] = _dot(a, wa2_ref[...]) + ba2_ref[...]
    cv = jnp.maximum(_dot(pooled, wc1_ref[...]) + bc1_ref[...], 0.0)
    val_ref[...] = _dot(cv, wc2_ref[...]) + bc2_ref[...]


def kernel(x, edge_index, batch, W1l, b1l, W1r, W2l, b2l, W2r,
           Wa1, ba1, Wa2, ba2, Wc1, bc1, Wc2, bc2):
    N, F = x.shape
    E = edge_index.shape[1]
    H = W1l.shape[0]
    A = Wa2.shape[0]
    NDUM = 240
    N8 = N + NDUM
    NW = _N_CORES * _N_SUB
    NCH = -(-E // (NW * _C * 5)) * 5
    EPAD = NW * NCH * _C

    f32 = jnp.float32
    i32 = jnp.int32
    pad = N + jnp.arange(EPAD - E, dtype=i32) % NDUM
    src3 = jnp.concatenate([edge_index[0], pad]).reshape(NW, NCH, _C)
    dst3 = jnp.concatenate([edge_index[1], pad]).reshape(NW, NCH, _C)
    x8 = jnp.pad(x, ((0, N8 - N), (0, 0)))
    batch8 = jnp.pad(batch, (0, N8 - N), constant_values=_G)

    u1, r1 = pl.pallas_call(
        _tc_pre_body,
        out_shape=[jax.ShapeDtypeStruct((N8, H), f32),
                   jax.ShapeDtypeStruct((N8, H), f32)],
        compiler_params=_TC_PARAMS,
    )(x8, W1l.T, W1r.T)

    s1, c1 = _make_sc_pass(N8, NCH, H, True)(u1, src3, dst3)

    h1, r2a = pl.pallas_call(
        functools.partial(_tc_mid_body, N),
        out_shape=[jax.ShapeDtypeStruct((N8, H), f32),
                   jax.ShapeDtypeStruct((N8, 2 * H), f32)],
        compiler_params=_TC_PARAMS,
    )(s1, c1, r1, b1l.reshape(1, H), W2r.T)

    s2, = _make_sc_pass(N8, NCH, H, False)(h1, src3, dst3)

    mu, value = pl.pallas_call(
        _tc_post_body,
        out_shape=[jax.ShapeDtypeStruct((_G, A), f32),
                   jax.ShapeDtypeStruct((_G, 1), f32)],
        compiler_params=_TC_PARAMS,
    )(s2, r2a, b2l.reshape(1, H), batch8.reshape(1, N8), W2l.T,
      Wa1.T, ba1.reshape(1, H), Wa2.T, ba2.reshape(1, A),
      Wc1.T, bc1.reshape(1, H), Wc2.T, bc2.reshape(1, 1))

    return (mu, value)
